# baseline (speedup 1.0000x reference)
import functools

import jax
import jax.numpy as jnp
from jax import lax
from jax.experimental import pallas as pl
from jax.experimental.pallas import tpu as pltpu
from jax.experimental.pallas import tpu_sc as plsc

N = 10000
E = 320000
D = 128
EAW = 32
NC = 2
NS = 16
NW = NC * NS
CHUNK = 128
NCH = 80
EPT = NCH * CHUNK
EPAD = NW * EPT
NPAD = 10240
ZROWS = NPAD // NS
OROWS = N // NS


_MESH = plsc.VectorSubcoreMesh(core_axis_name="c", subcore_axis_name="s")


def _zero_fill(buf, width):
  z16 = jnp.zeros((16,), jnp.float32)

  def zrow(i, _):
    for k in range(width // 16):
      buf[i, pl.ds(k * 16, 16)] = z16
    return 0

  lax.fori_loop(0, CHUNK, zrow, 0)


def _unpack(pk, sidx, didx, j, t):
  m = jnp.full((16,), 0xFFFF, jnp.int32)
  for k in range(CHUNK // 16):
    v = pk[j, pl.ds(k * 16, 16)]
    sidx[t, pl.ds(k * 16, 16)] = v & m
    didx[t, pl.ds(k * 16, 16)] = lax.shift_right_logical(v, 16)


def _zero_acc(src_rows, acc, s):
  _zero_fill(src_rows, D)
  for k in range(ZROWS // CHUNK):
    pltpu.sync_copy(src_rows, acc.at[pl.ds(s * ZROWS + k * CHUNK, CHUNK)])
  plsc.subcore_barrier()


def _spmm_body(pk_h, x_h, s_out, pk, sidx, didx, rows0, rows1, acc, gsem, ssem):
  c = lax.axis_index("c")
  s = lax.axis_index("s")
  tid = c * NS + s

  pltpu.sync_copy(pk_h.at[tid], pk)
  _zero_acc(rows0, acc, s)

  _unpack(pk, sidx, didx, 0, 0)
  pltpu.async_copy(x_h.at[sidx.at[0]], rows0, gsem)
  bufs = (rows0, rows1)

  def pair(g, _):
    for t in range(2):
      j = 2 * g + t
      rb = bufs[t]
      ob = bufs[1 - t]
      pltpu.make_async_copy(x_h.at[sidx.at[t]], rb, gsem).wait()

      @pl.when(j >= 1)
      def _wait_prev():
        pltpu.make_async_copy(ob, acc.at[didx.at[1 - t]], ssem).wait()

      pltpu.async_copy(rb, acc.at[didx.at[t]], ssem, add=True)

      @pl.when(j + 1 < NCH)
      def _next_gather():
        _unpack(pk, sidx, didx, j + 1, 1 - t)
        pltpu.async_copy(x_h.at[sidx.at[1 - t]], ob, gsem)
    return 0

  lax.fori_loop(0, NCH // 2, pair, 0)
  pltpu.make_async_copy(rows1, acc.at[didx.at[1]], ssem).wait()
  plsc.subcore_barrier()

  pltpu.sync_copy(acc.at[pl.ds(s * ZROWS, ZROWS)],
                  s_out.at[c, pl.ds(s * ZROWS, ZROWS)])


_spmm = pl.kernel(
    _spmm_body, mesh=_MESH,
    out_type=[jax.ShapeDtypeStruct((NC, NPAD, D), jnp.float32)],
    scratch_types=[
        pltpu.VMEM((NCH, CHUNK), jnp.int32),
        pltpu.VMEM((2, CHUNK), jnp.int32),
        pltpu.VMEM((2, CHUNK), jnp.int32),
        pltpu.VMEM((CHUNK, D), jnp.float32),
        pltpu.VMEM((CHUNK, D), jnp.float32),
        pltpu.VMEM_SHARED((NPAD, D), jnp.float32),
        pltpu.SemaphoreType.DMA,
        pltpu.SemaphoreType.DMA,
    ])


def _unpack_d(pk, didx, j, t):
  for k in range(CHUNK // 16):
    v = pk[j, pl.ds(k * 16, 16)]
    didx[t, pl.ds(k * 16, 16)] = lax.shift_right_logical(v, 16)


def _ea_body(pk_h, ea_h, ea_out, pk, didx, eb0, eb1, rows0, rows1,
             eacc, gsem, ssem):
  c = lax.axis_index("c")
  s = lax.axis_index("s")
  tid = c * NS + s

  pltpu.sync_copy(pk_h.at[tid], pk)
  _zero_fill(rows0, D)
  for k in range(ZROWS // CHUNK):
    pltpu.sync_copy(rows0, eacc.at[pl.ds(s * ZROWS + k * CHUNK, CHUNK)])
  _zero_fill(rows1, D)
  one0 = jnp.where(lax.iota(jnp.int32, 16) == 0,
                   jnp.float32(1), jnp.float32(0))

  def fill_tail(i, _):
    rows0[i, pl.ds(16, 16)] = one0
    rows1[i, pl.ds(16, 16)] = one0
    return 0

  lax.fori_loop(0, CHUNK, fill_tail, 0)
  plsc.subcore_barrier()

  _unpack_d(pk, didx, 0, 0)
  pltpu.async_copy(ea_h.at[tid, 0], eb0, gsem)
  ebufs = (eb0, eb1)
  rbufs = (rows0, rows1)

  def pair(g, _):
    for t in range(2):
      j = 2 * g + t
      eb = ebufs[t]
      rb = rbufs[t]
      pltpu.make_async_copy(ea_h.at[tid, j], eb, gsem).wait()

      @pl.when(j >= 2)
      def _wait_prev():
        pltpu.make_async_copy(rb, eacc.at[didx.at[t]], ssem).wait()

      @pl.when(j + 1 < NCH)
      def _next_load():
        pltpu.async_copy(ea_h.at[tid, j + 1], ebufs[1 - t], gsem)

      _unpack_d(pk, didx, j, t)

      def expand(r, _):
        for q in range(8):
          rb[r * 8 + q, pl.ds(0, 16)] = eb[r, pl.ds(q * 16, 16)]
        return 0

      lax.fori_loop(0, 16, expand, 0)
      pltpu.async_copy(rb, eacc.at[didx.at[t]], ssem, add=True)
    return 0

  lax.fori_loop(0, NCH // 2, pair, 0)
  pltpu.make_async_copy(rows0, eacc.at[didx.at[0]], ssem).wait()
  pltpu.make_async_copy(rows1, eacc.at[didx.at[1]], ssem).wait()
  plsc.subcore_barrier()

  orows = ZROWS * EAW // 128
  for k in range(ZROWS // CHUNK):
    pltpu.sync_copy(eacc.at[pl.ds(s * ZROWS + k * CHUNK, CHUNK)], rows0)

    def repack(r, _):
      for q in range(8):
        rows1[r, pl.ds(q * 16, 16)] = rows0[r * 4 + q // 2,
                                            pl.ds((q % 2) * 16, 16)]
      return 0

    lax.fori_loop(0, CHUNK * EAW // 128, repack, 0)
    pltpu.sync_copy(rows1.at[pl.ds(0, CHUNK * EAW // 128)],
                    ea_out.at[c, pl.ds(s * orows + k * (CHUNK * EAW // 128),
                                       CHUNK * EAW // 128)])


_ea_scatter = pl.kernel(
    _ea_body, mesh=_MESH,
    out_type=[jax.ShapeDtypeStruct((NC, NPAD * EAW // 128, 128), jnp.float32)],
    scratch_types=[
        pltpu.VMEM((NCH, CHUNK), jnp.int32),
        pltpu.VMEM((2, CHUNK), jnp.int32),
        pltpu.VMEM((16, CHUNK), jnp.float32),
        pltpu.VMEM((16, CHUNK), jnp.float32),
        pltpu.VMEM((CHUNK, D), jnp.float32),
        pltpu.VMEM((CHUNK, D), jnp.float32),
        pltpu.VMEM_SHARED((NPAD, D), jnp.float32),
        pltpu.SemaphoreType.DMA,
        pltpu.SemaphoreType.DMA,
    ])


_BLK = 1000
_GRID = N // _BLK


def _dense_mid_body(h, s0, s1, e0, e1, wm, bm, wu, bu, out):
  sp = s0[0] + s1[0]
  eas = e0[0] + e1[0]
  agg = (jnp.dot(sp, wm[:D, :], preferred_element_type=jnp.float32)
         + jnp.dot(eas[:, :16], wm[D:, :], preferred_element_type=jnp.float32)
         + eas[:, 16:17] * bm[...])
  upd = (jnp.dot(h[...], wu[:D, :], preferred_element_type=jnp.float32)
         + jnp.dot(agg, wu[D:, :], preferred_element_type=jnp.float32)
         + bu[...])
  out[...] = jnp.maximum(upd, 0.0)


def _dense_final_body(h, s0, s1, e0, e1, wm, bm, wu, bu, wp1, bp1, wp2, bp2,
                      z_out, gr_out, accum):
  i = pl.program_id(0)

  @pl.when(i == 0)
  def _init():
    accum[...] = jnp.zeros_like(accum)

  sp = s0[0] + s1[0]
  eas = e0[0] + e1[0]
  agg = (jnp.dot(sp, wm[:D, :], preferred_element_type=jnp.float32)
         + jnp.dot(eas[:, :16], wm[D:, :], preferred_element_type=jnp.float32)
         + eas[:, 16:17] * bm[...])
  upd = (jnp.dot(h[...], wu[:D, :], preferred_element_type=jnp.float32)
         + jnp.dot(agg, wu[D:, :], preferred_element_type=jnp.float32)
         + bu[...])
  hb = jnp.maximum(upd, 0.0)
  accum[...] += jnp.sum(hb, axis=0, keepdims=True)

  @pl.when(i == _GRID - 1)
  def _readout():
    gr = accum[...] * (1.0 / N)
    gr_out[...] = gr
    t = jnp.maximum(
        jnp.dot(gr, wp1[...], preferred_element_type=jnp.float32) + bp1[...],
        0.0)
    z_out[...] = jnp.dot(t, wp2[...], preferred_element_type=jnp.float32) + bp2[...]


def _row_spec(width):
  return pl.BlockSpec((_BLK, width), lambda i: (i, 0))


def _full_spec(shape):
  return pl.BlockSpec(shape, lambda i: tuple(0 for _ in shape))


def _dense_mid(h, s_parts, ea_parts, wm, bm, wu, bu):
  return pl.pallas_call(
      _dense_mid_body,
      grid=(_GRID,),
      in_specs=[
          _row_spec(D),
          pl.BlockSpec((1, _BLK, D), lambda i: (0, i, 0)),
          pl.BlockSpec((1, _BLK, D), lambda i: (1, i, 0)),
          pl.BlockSpec((1, _BLK, EAW), lambda i: (0, i, 0)),
          pl.BlockSpec((1, _BLK, EAW), lambda i: (1, i, 0)),
          _full_spec((D + 16, D)),
          _full_spec((1, D)),
          _full_spec((2 * D, D)),
          _full_spec((1, D)),
      ],
      out_specs=_row_spec(D),
      out_shape=jax.ShapeDtypeStruct((N, D), jnp.float32),
  )(h, s_parts, s_parts, ea_parts, ea_parts, wm, bm, wu, bu)


def _dense_final(h, s_parts, ea_parts, wm, bm, wu, bu, wp1, bp1, wp2, bp2):
  return pl.pallas_call(
      _dense_final_body,
      grid=(_GRID,),
      in_specs=[
          _row_spec(D),
          pl.BlockSpec((1, _BLK, D), lambda i: (0, i, 0)),
          pl.BlockSpec((1, _BLK, D), lambda i: (1, i, 0)),
          pl.BlockSpec((1, _BLK, EAW), lambda i: (0, i, 0)),
          pl.BlockSpec((1, _BLK, EAW), lambda i: (1, i, 0)),
          _full_spec((D + 16, D)),
          _full_spec((1, D)),
          _full_spec((2 * D, D)),
          _full_spec((1, D)),
          _full_spec((D, D)),
          _full_spec((1, D)),
          _full_spec((D, D)),
          _full_spec((1, D)),
      ],
      out_specs=[_full_spec((1, D)), _full_spec((1, D))],
      out_shape=[jax.ShapeDtypeStruct((1, D), jnp.float32),
                 jax.ShapeDtypeStruct((1, D), jnp.float32)],
      scratch_shapes=[pltpu.VMEM((1, D), jnp.float32)],
  )(h, s_parts, s_parts, ea_parts, ea_parts, wm, bm, wu, bu,
    wp1, bp1, wp2, bp2)


def kernel(x, edge_index, edge_attr,
           W_msg0, b_msg0, W_upd0, b_upd0,
           W_msg1, b_msg1, W_upd1, b_upd1,
           W_msg2, b_msg2, W_upd2, b_upd2,
           W_p1, b_p1, W_p2, b_p2):
  src = edge_index[0].astype(jnp.int32)
  dst = edge_index[1].astype(jnp.int32)
  pad = EPAD - E
  ar = lax.iota(jnp.int32, pad)
  src_p = jnp.concatenate([src, ar % N])
  dst_p = jnp.concatenate([dst, N + ar % (NPAD - N)])
  packed3 = (src_p | (dst_p << 16)).reshape(NW, NCH, CHUNK)
  ea4 = jnp.concatenate(
      [edge_attr, jnp.zeros((pad, 16), jnp.float32)]).reshape(
          NW, NCH, 16, CHUNK)

  bm0 = b_msg0.reshape(1, D); bu0 = b_upd0.reshape(1, D)
  bm1 = b_msg1.reshape(1, D); bu1 = b_upd1.reshape(1, D)
  bm2 = b_msg2.reshape(1, D); bu2 = b_upd2.reshape(1, D)
  bp1 = b_p1.reshape(1, D); bp2 = b_p2.reshape(1, D)

  (ea_flat,) = _ea_scatter(packed3, ea4)
  ea_parts = ea_flat.reshape(NC, NPAD, EAW)
  (s_parts,) = _spmm(packed3, x)
  h = _dense_mid(x, s_parts, ea_parts, W_msg0, bm0, W_upd0, bu0)
  (s_parts2,) = _spmm(packed3, h)
  h = _dense_mid(h, s_parts2, ea_parts, W_msg1, bm1, W_upd1, bu1)
  (s_parts3,) = _spmm(packed3, h)
  z, gr = _dense_final(h, s_parts3, ea_parts, W_msg2, bm2, W_upd2, bu2,
                       W_p1, bp1, W_p2, bp2)
  return (z.reshape(D), gr.reshape(D))


# speedup vs baseline: 1.1418x; 1.1418x over previous
import functools

import jax
import jax.numpy as jnp
from jax import lax
from jax.experimental import pallas as pl
from jax.experimental.pallas import tpu as pltpu
from jax.experimental.pallas import tpu_sc as plsc

N = 10000
E = 320000
D = 128
EAW = 32
NC = 2
NS = 16
NW = NC * NS
CHUNK = 128
NCH = 80
EPT = NCH * CHUNK
EPAD = NW * EPT
NPAD = 10240
ZROWS = NPAD // NS
OROWS = N // NS


_MESH = plsc.VectorSubcoreMesh(core_axis_name="c", subcore_axis_name="s")


def _zero_fill(buf, width):
  z16 = jnp.zeros((16,), jnp.float32)

  def zrow(i, _):
    for k in range(width // 16):
      buf[i, pl.ds(k * 16, 16)] = z16
    return 0

  lax.fori_loop(0, CHUNK, zrow, 0)


def _unpack(pk, sidx, didx, j, t):
  m = jnp.full((16,), 0xFFFF, jnp.int32)
  for k in range(CHUNK // 16):
    v = pk[j, pl.ds(k * 16, 16)]
    sidx[t, pl.ds(k * 16, 16)] = v & m
    didx[t, pl.ds(k * 16, 16)] = lax.shift_right_logical(v, 16)


def _zero_acc(src_rows, acc, s):
  _zero_fill(src_rows, D)
  for k in range(ZROWS // CHUNK):
    pltpu.sync_copy(src_rows, acc.at[pl.ds(s * ZROWS + k * CHUNK, CHUNK)])
  plsc.subcore_barrier()


def _spmm_body(pk_h, x_h, s_out, pk, sidx, didx, rows0, rows1, acc, gsem, ssem):
  c = lax.axis_index("c")
  s = lax.axis_index("s")
  tid = c * NS + s

  pltpu.sync_copy(pk_h.at[tid], pk)
  _zero_acc(rows0, acc, s)

  _unpack(pk, sidx, didx, 0, 0)
  pltpu.async_copy(x_h.at[sidx.at[0]], rows0, gsem)
  bufs = (rows0, rows1)

  def pair(g, _):
    for t in range(2):
      j = 2 * g + t
      rb = bufs[t]
      ob = bufs[1 - t]
      pltpu.make_async_copy(x_h.at[sidx.at[t]], rb, gsem).wait()

      @pl.when(j >= 1)
      def _wait_prev():
        pltpu.make_async_copy(ob, acc.at[didx.at[1 - t]], ssem).wait()

      pltpu.async_copy(rb, acc.at[didx.at[t]], ssem, add=True)

      @pl.when(j + 1 < NCH)
      def _next_gather():
        _unpack(pk, sidx, didx, j + 1, 1 - t)
        pltpu.async_copy(x_h.at[sidx.at[1 - t]], ob, gsem)
    return 0

  lax.fori_loop(0, NCH // 2, pair, 0)
  pltpu.make_async_copy(rows1, acc.at[didx.at[1]], ssem).wait()
  plsc.subcore_barrier()

  pltpu.sync_copy(acc.at[pl.ds(s * ZROWS, ZROWS)],
                  s_out.at[c, pl.ds(s * ZROWS, ZROWS)])


_spmm = pl.kernel(
    _spmm_body, mesh=_MESH,
    out_type=[jax.ShapeDtypeStruct((NC, NPAD, D), jnp.float32)],
    scratch_types=[
        pltpu.VMEM((NCH, CHUNK), jnp.int32),
        pltpu.VMEM((2, CHUNK), jnp.int32),
        pltpu.VMEM((2, CHUNK), jnp.int32),
        pltpu.VMEM((CHUNK, D), jnp.float32),
        pltpu.VMEM((CHUNK, D), jnp.float32),
        pltpu.VMEM_SHARED((NPAD, D), jnp.float32),
        pltpu.SemaphoreType.DMA,
        pltpu.SemaphoreType.DMA,
    ])


def _unpack_d(pk, didx, j, t):
  for k in range(CHUNK // 16):
    v = pk[j, pl.ds(k * 16, 16)]
    didx[t, pl.ds(k * 16, 16)] = lax.shift_right_logical(v, 16)


def _ea_body(pk_h, ea_h, ea_out, pk, didx, eb0, eb1, rows0, rows1,
             eacc, gsem, ssem):
  c = lax.axis_index("c")
  s = lax.axis_index("s")
  tid = c * NS + s

  pltpu.sync_copy(pk_h.at[tid], pk)
  _zero_fill(rows0, D)
  for k in range(ZROWS // CHUNK):
    pltpu.sync_copy(rows0, eacc.at[pl.ds(s * ZROWS + k * CHUNK, CHUNK)])
  _zero_fill(rows1, D)
  one0 = jnp.where(lax.iota(jnp.int32, 16) == 0,
                   jnp.float32(1), jnp.float32(0))

  def fill_tail(i, _):
    rows0[i, pl.ds(16, 16)] = one0
    rows1[i, pl.ds(16, 16)] = one0
    return 0

  lax.fori_loop(0, CHUNK, fill_tail, 0)
  plsc.subcore_barrier()

  _unpack_d(pk, didx, 0, 0)
  pltpu.async_copy(ea_h.at[tid, 0], eb0, gsem)
  ebufs = (eb0, eb1)
  rbufs = (rows0, rows1)

  def pair(g, _):
    for t in range(2):
      j = 2 * g + t
      eb = ebufs[t]
      rb = rbufs[t]
      pltpu.make_async_copy(ea_h.at[tid, j], eb, gsem).wait()

      @pl.when(j >= 2)
      def _wait_prev():
        pltpu.make_async_copy(rb, eacc.at[didx.at[t]], ssem).wait()

      @pl.when(j + 1 < NCH)
      def _next_load():
        pltpu.async_copy(ea_h.at[tid, j + 1], ebufs[1 - t], gsem)

      _unpack_d(pk, didx, j, t)

      def expand(r, _):
        for q in range(8):
          rb[r * 8 + q, pl.ds(0, 16)] = eb[r, pl.ds(q * 16, 16)]
        return 0

      lax.fori_loop(0, 16, expand, 0)
      pltpu.async_copy(rb, eacc.at[didx.at[t]], ssem, add=True)
    return 0

  lax.fori_loop(0, NCH // 2, pair, 0)
  pltpu.make_async_copy(rows0, eacc.at[didx.at[0]], ssem).wait()
  pltpu.make_async_copy(rows1, eacc.at[didx.at[1]], ssem).wait()
  plsc.subcore_barrier()

  pltpu.sync_copy(eacc.at[pl.ds(s * ZROWS, ZROWS)],
                  ea_out.at[c, pl.ds(s * ZROWS, ZROWS)])


_ea_scatter = pl.kernel(
    _ea_body, mesh=_MESH,
    out_type=[jax.ShapeDtypeStruct((NC, NPAD, D), jnp.float32)],
    scratch_types=[
        pltpu.VMEM((NCH, CHUNK), jnp.int32),
        pltpu.VMEM((2, CHUNK), jnp.int32),
        pltpu.VMEM((16, CHUNK), jnp.float32),
        pltpu.VMEM((16, CHUNK), jnp.float32),
        pltpu.VMEM((CHUNK, D), jnp.float32),
        pltpu.VMEM((CHUNK, D), jnp.float32),
        pltpu.VMEM_SHARED((NPAD, D), jnp.float32),
        pltpu.SemaphoreType.DMA,
        pltpu.SemaphoreType.DMA,
    ])


_BLK = 1000
_GRID = N // _BLK


def _dense_mid_body(h, s0, s1, e0, e1, wm, bm, wu, bu, out):
  sp = s0[0] + s1[0]
  eas = e0[0] + e1[0]
  agg = (jnp.dot(sp, wm[:D, :], preferred_element_type=jnp.float32)
         + jnp.dot(eas[:, :16], wm[D:, :], preferred_element_type=jnp.float32)
         + eas[:, 16:17] * bm[...])
  upd = (jnp.dot(h[...], wu[:D, :], preferred_element_type=jnp.float32)
         + jnp.dot(agg, wu[D:, :], preferred_element_type=jnp.float32)
         + bu[...])
  out[...] = jnp.maximum(upd, 0.0)


def _dense_final_body(h, s0, s1, e0, e1, wm, bm, wu, bu, wp1, bp1, wp2, bp2,
                      z_out, gr_out, accum):
  i = pl.program_id(0)

  @pl.when(i == 0)
  def _init():
    accum[...] = jnp.zeros_like(accum)

  sp = s0[0] + s1[0]
  eas = e0[0] + e1[0]
  agg = (jnp.dot(sp, wm[:D, :], preferred_element_type=jnp.float32)
         + jnp.dot(eas[:, :16], wm[D:, :], preferred_element_type=jnp.float32)
         + eas[:, 16:17] * bm[...])
  upd = (jnp.dot(h[...], wu[:D, :], preferred_element_type=jnp.float32)
         + jnp.dot(agg, wu[D:, :], preferred_element_type=jnp.float32)
         + bu[...])
  hb = jnp.maximum(upd, 0.0)
  accum[...] += jnp.sum(hb, axis=0, keepdims=True)

  @pl.when(i == _GRID - 1)
  def _readout():
    gr = accum[...] * (1.0 / N)
    gr_out[...] = gr
    t = jnp.maximum(
        jnp.dot(gr, wp1[...], preferred_element_type=jnp.float32) + bp1[...],
        0.0)
    z_out[...] = jnp.dot(t, wp2[...], preferred_element_type=jnp.float32) + bp2[...]


def _row_spec(width):
  return pl.BlockSpec((_BLK, width), lambda i: (i, 0))


def _full_spec(shape):
  return pl.BlockSpec(shape, lambda i: tuple(0 for _ in shape))


def _dense_mid(h, s_parts, ea_parts, wm, bm, wu, bu):
  return pl.pallas_call(
      _dense_mid_body,
      grid=(_GRID,),
      in_specs=[
          _row_spec(D),
          pl.BlockSpec((1, _BLK, D), lambda i: (0, i, 0)),
          pl.BlockSpec((1, _BLK, D), lambda i: (1, i, 0)),
          pl.BlockSpec((1, _BLK, D), lambda i: (0, i, 0)),
          pl.BlockSpec((1, _BLK, D), lambda i: (1, i, 0)),
          _full_spec((D + 16, D)),
          _full_spec((1, D)),
          _full_spec((2 * D, D)),
          _full_spec((1, D)),
      ],
      out_specs=_row_spec(D),
      out_shape=jax.ShapeDtypeStruct((N, D), jnp.float32),
  )(h, s_parts, s_parts, ea_parts, ea_parts, wm, bm, wu, bu)


def _dense_final(h, s_parts, ea_parts, wm, bm, wu, bu, wp1, bp1, wp2, bp2):
  return pl.pallas_call(
      _dense_final_body,
      grid=(_GRID,),
      in_specs=[
          _row_spec(D),
          pl.BlockSpec((1, _BLK, D), lambda i: (0, i, 0)),
          pl.BlockSpec((1, _BLK, D), lambda i: (1, i, 0)),
          pl.BlockSpec((1, _BLK, D), lambda i: (0, i, 0)),
          pl.BlockSpec((1, _BLK, D), lambda i: (1, i, 0)),
          _full_spec((D + 16, D)),
          _full_spec((1, D)),
          _full_spec((2 * D, D)),
          _full_spec((1, D)),
          _full_spec((D, D)),
          _full_spec((1, D)),
          _full_spec((D, D)),
          _full_spec((1, D)),
      ],
      out_specs=[_full_spec((1, D)), _full_spec((1, D))],
      out_shape=[jax.ShapeDtypeStruct((1, D), jnp.float32),
                 jax.ShapeDtypeStruct((1, D), jnp.float32)],
      scratch_shapes=[pltpu.VMEM((1, D), jnp.float32)],
  )(h, s_parts, s_parts, ea_parts, ea_parts, wm, bm, wu, bu,
    wp1, bp1, wp2, bp2)


def kernel(x, edge_index, edge_attr,
           W_msg0, b_msg0, W_upd0, b_upd0,
           W_msg1, b_msg1, W_upd1, b_upd1,
           W_msg2, b_msg2, W_upd2, b_upd2,
           W_p1, b_p1, W_p2, b_p2):
  src = edge_index[0].astype(jnp.int32)
  dst = edge_index[1].astype(jnp.int32)
  pad = EPAD - E
  ar = lax.iota(jnp.int32, pad)
  src_p = jnp.concatenate([src, ar % N])
  dst_p = jnp.concatenate([dst, N + ar % (NPAD - N)])
  packed3 = (src_p | (dst_p << 16)).reshape(NW, NCH, CHUNK)
  ea4 = jnp.concatenate(
      [edge_attr, jnp.zeros((pad, 16), jnp.float32)]).reshape(
          NW, NCH, 16, CHUNK)

  bm0 = b_msg0.reshape(1, D); bu0 = b_upd0.reshape(1, D)
  bm1 = b_msg1.reshape(1, D); bu1 = b_upd1.reshape(1, D)
  bm2 = b_msg2.reshape(1, D); bu2 = b_upd2.reshape(1, D)
  bp1 = b_p1.reshape(1, D); bp2 = b_p2.reshape(1, D)

  (ea_parts,) = _ea_scatter(packed3, ea4)
  (s_parts,) = _spmm(packed3, x)
  h = _dense_mid(x, s_parts, ea_parts, W_msg0, bm0, W_upd0, bu0)
  (s_parts2,) = _spmm(packed3, h)
  h = _dense_mid(h, s_parts2, ea_parts, W_msg1, bm1, W_upd1, bu1)
  (s_parts3,) = _spmm(packed3, h)
  z, gr = _dense_final(h, s_parts3, ea_parts, W_msg2, bm2, W_upd2, bu2,
                       W_p1, bp1, W_p2, bp2)
  return (z.reshape(D), gr.reshape(D))


# speedup vs baseline: 1.1736x; 1.0279x over previous
import functools

import jax
import jax.numpy as jnp
from jax import lax
from jax.experimental import pallas as pl
from jax.experimental.pallas import tpu as pltpu
from jax.experimental.pallas import tpu_sc as plsc

N = 10000
E = 320000
D = 128
EAW = 32
NC = 2
NS = 16
NW = NC * NS
CHUNK = 128
NCH = 80
EPT = NCH * CHUNK
EPAD = NW * EPT
NPAD = 10240
ZROWS = NPAD // NS
OROWS = N // NS


_MESH = plsc.VectorSubcoreMesh(core_axis_name="c", subcore_axis_name="s")


def _zero_fill(buf, width):
  z16 = jnp.zeros((16,), jnp.float32)

  def zrow(i, _):
    for k in range(width // 16):
      buf[i, pl.ds(k * 16, 16)] = z16
    return 0

  lax.fori_loop(0, CHUNK, zrow, 0)


def _unpack(pk, sidx, didx, j, t):
  m = jnp.full((16,), 0xFFFF, jnp.int32)
  for k in range(CHUNK // 16):
    v = pk[j, pl.ds(k * 16, 16)]
    sidx[t, pl.ds(k * 16, 16)] = v & m
    didx[t, pl.ds(k * 16, 16)] = lax.shift_right_logical(v, 16)


def _zero_acc(src_rows, acc, s):
  _zero_fill(src_rows, D)
  for k in range(ZROWS // CHUNK):
    pltpu.sync_copy(src_rows, acc.at[pl.ds(s * ZROWS + k * CHUNK, CHUNK)])
  plsc.subcore_barrier()


def _spmm_body(pk_h, x_h, s_out, pk, sidx, didx, rows0, rows1, acc, gsem, ssem):
  c = lax.axis_index("c")
  s = lax.axis_index("s")
  tid = c * NS + s

  pltpu.sync_copy(pk_h.at[tid], pk)
  _zero_acc(rows0, acc, s)

  _unpack(pk, sidx, didx, 0, 0)
  pltpu.async_copy(x_h.at[sidx.at[0]], rows0, gsem)
  bufs = (rows0, rows1)

  def pair(g, _):
    for t in range(2):
      j = 2 * g + t
      rb = bufs[t]
      ob = bufs[1 - t]
      pltpu.make_async_copy(x_h.at[sidx.at[t]], rb, gsem).wait()

      @pl.when(j >= 1)
      def _wait_prev():
        pltpu.make_async_copy(ob, acc.at[didx.at[1 - t]], ssem).wait()

      pltpu.async_copy(rb, acc.at[didx.at[t]], ssem, add=True)

      @pl.when(j + 1 < NCH)
      def _next_gather():
        _unpack(pk, sidx, didx, j + 1, 1 - t)
        pltpu.async_copy(x_h.at[sidx.at[1 - t]], ob, gsem)
    return 0

  lax.fori_loop(0, NCH // 2, pair, 0)
  pltpu.make_async_copy(rows1, acc.at[didx.at[1]], ssem).wait()
  plsc.subcore_barrier()

  pltpu.sync_copy(acc.at[pl.ds(s * ZROWS, ZROWS)],
                  s_out.at[c, pl.ds(s * ZROWS, ZROWS)])


_spmm = pl.kernel(
    _spmm_body, mesh=_MESH,
    out_type=[jax.ShapeDtypeStruct((NC, NPAD, D), jnp.float32)],
    scratch_types=[
        pltpu.VMEM((NCH, CHUNK), jnp.int32),
        pltpu.VMEM((2, CHUNK), jnp.int32),
        pltpu.VMEM((2, CHUNK), jnp.int32),
        pltpu.VMEM((CHUNK, D), jnp.float32),
        pltpu.VMEM((CHUNK, D), jnp.float32),
        pltpu.VMEM_SHARED((NPAD, D), jnp.float32),
        pltpu.SemaphoreType.DMA,
        pltpu.SemaphoreType.DMA,
    ])


def _unpack_d(pk, didx, j, t):
  for k in range(CHUNK // 16):
    v = pk[j, pl.ds(k * 16, 16)]
    didx[t, pl.ds(k * 16, 16)] = lax.shift_right_logical(v, 16)


ECH = E // CHUNK


def _l0_body(pk_h, x_h, ea_h, s_out, ea_out,
             pk, sidx, didx, eb0, eb1, rows0, rows1, acc, gsem, ssem):
  c = lax.axis_index("c")
  s = lax.axis_index("s")
  tid = c * NS + s

  pltpu.sync_copy(pk_h.at[tid], pk)

  _zero_acc(rows0, acc, s)
  _zero_fill(rows1, D)
  one0 = jnp.where(lax.iota(jnp.int32, 16) == 0,
                   jnp.float32(1), jnp.float32(0))

  def fill_tail(i, _):
    rows0[i, pl.ds(16, 16)] = one0
    rows1[i, pl.ds(16, 16)] = one0
    return 0

  lax.fori_loop(0, CHUNK, fill_tail, 0)

  nj = jnp.minimum(ECH - tid * NCH, NCH)
  _unpack_d(pk, didx, 0, 0)
  pltpu.async_copy(ea_h.at[tid * NCH], eb0, gsem)
  ebufs = (eb0, eb1)
  rbufs = (rows0, rows1)

  def ea_pair(g, _):
    for t in range(2):
      j = 2 * g + t
      eb = ebufs[t]
      rb = rbufs[t]
      pltpu.make_async_copy(ea_h.at[tid * NCH + j], eb, gsem).wait()

      @pl.when(j >= 2)
      def _wait_prev():
        pltpu.make_async_copy(rb, acc.at[didx.at[t]], ssem).wait()

      @pl.when(j + 1 < nj)
      def _next_load():
        pltpu.async_copy(ea_h.at[tid * NCH + j + 1], ebufs[1 - t], gsem)

      _unpack_d(pk, didx, j, t)

      def expand(r, _):
        for q in range(8):
          rb[r * 8 + q, pl.ds(0, 16)] = eb[r, pl.ds(q * 16, 16)]
        return 0

      lax.fori_loop(0, 16, expand, 0)
      pltpu.async_copy(rb, acc.at[didx.at[t]], ssem, add=True)
    return 0

  lax.fori_loop(0, nj // 2, ea_pair, 0)
  pltpu.make_async_copy(rows0, acc.at[didx.at[0]], ssem).wait()
  pltpu.make_async_copy(rows1, acc.at[didx.at[1]], ssem).wait()
  plsc.subcore_barrier()
  pltpu.sync_copy(acc.at[pl.ds(s * ZROWS, ZROWS)],
                  ea_out.at[c, pl.ds(s * ZROWS, ZROWS)])

  _zero_acc(rows0, acc, s)
  _unpack(pk, sidx, didx, 0, 0)
  pltpu.async_copy(x_h.at[sidx.at[0]], rows0, gsem)

  def pair(g, _):
    for t in range(2):
      j = 2 * g + t
      rb = rbufs[t]
      ob = rbufs[1 - t]
      pltpu.make_async_copy(x_h.at[sidx.at[t]], rb, gsem).wait()

      @pl.when(j >= 1)
      def _wait_prev():
        pltpu.make_async_copy(ob, acc.at[didx.at[1 - t]], ssem).wait()

      pltpu.async_copy(rb, acc.at[didx.at[t]], ssem, add=True)

      @pl.when(j + 1 < NCH)
      def _next_gather():
        _unpack(pk, sidx, didx, j + 1, 1 - t)
        pltpu.async_copy(x_h.at[sidx.at[1 - t]], ob, gsem)
    return 0

  lax.fori_loop(0, NCH // 2, pair, 0)
  pltpu.make_async_copy(rows1, acc.at[didx.at[1]], ssem).wait()
  plsc.subcore_barrier()
  pltpu.sync_copy(acc.at[pl.ds(s * ZROWS, ZROWS)],
                  s_out.at[c, pl.ds(s * ZROWS, ZROWS)])


_spmm_l0 = pl.kernel(
    _l0_body, mesh=_MESH,
    out_type=[jax.ShapeDtypeStruct((NC, NPAD, D), jnp.float32),
              jax.ShapeDtypeStruct((NC, NPAD, D), jnp.float32)],
    scratch_types=[
        pltpu.VMEM((NCH, CHUNK), jnp.int32),
        pltpu.VMEM((2, CHUNK), jnp.int32),
        pltpu.VMEM((2, CHUNK), jnp.int32),
        pltpu.VMEM((16, CHUNK), jnp.float32),
        pltpu.VMEM((16, CHUNK), jnp.float32),
        pltpu.VMEM((CHUNK, D), jnp.float32),
        pltpu.VMEM((CHUNK, D), jnp.float32),
        pltpu.VMEM_SHARED((NPAD, D), jnp.float32),
        pltpu.SemaphoreType.DMA,
        pltpu.SemaphoreType.DMA,
    ])


_BLK = 1000
_GRID = N // _BLK


def _dense_mid_body(h, s0, s1, e0, e1, wm, bm, wu, bu, out):
  sp = s0[0] + s1[0]
  eas = e0[0] + e1[0]
  agg = (jnp.dot(sp, wm[:D, :], preferred_element_type=jnp.float32)
         + jnp.dot(eas[:, :16], wm[D:, :], preferred_element_type=jnp.float32)
         + eas[:, 16:17] * bm[...])
  upd = (jnp.dot(h[...], wu[:D, :], preferred_element_type=jnp.float32)
         + jnp.dot(agg, wu[D:, :], preferred_element_type=jnp.float32)
         + bu[...])
  out[...] = jnp.maximum(upd, 0.0)


def _dense_final_body(h, s0, s1, e0, e1, wm, bm, wu, bu, wp1, bp1, wp2, bp2,
                      z_out, gr_out, accum):
  i = pl.program_id(0)

  @pl.when(i == 0)
  def _init():
    accum[...] = jnp.zeros_like(accum)

  sp = s0[0] + s1[0]
  eas = e0[0] + e1[0]
  agg = (jnp.dot(sp, wm[:D, :], preferred_element_type=jnp.float32)
         + jnp.dot(eas[:, :16], wm[D:, :], preferred_element_type=jnp.float32)
         + eas[:, 16:17] * bm[...])
  upd = (jnp.dot(h[...], wu[:D, :], preferred_element_type=jnp.float32)
         + jnp.dot(agg, wu[D:, :], preferred_element_type=jnp.float32)
         + bu[...])
  hb = jnp.maximum(upd, 0.0)
  accum[...] += jnp.sum(hb, axis=0, keepdims=True)

  @pl.when(i == _GRID - 1)
  def _readout():
    gr = accum[...] * (1.0 / N)
    gr_out[...] = gr
    t = jnp.maximum(
        jnp.dot(gr, wp1[...], preferred_element_type=jnp.float32) + bp1[...],
        0.0)
    z_out[...] = jnp.dot(t, wp2[...], preferred_element_type=jnp.float32) + bp2[...]


def _row_spec(width):
  return pl.BlockSpec((_BLK, width), lambda i: (i, 0))


def _full_spec(shape):
  return pl.BlockSpec(shape, lambda i: tuple(0 for _ in shape))


def _dense_mid(h, s_parts, ea_parts, wm, bm, wu, bu):
  return pl.pallas_call(
      _dense_mid_body,
      grid=(_GRID,),
      in_specs=[
          _row_spec(D),
          pl.BlockSpec((1, _BLK, D), lambda i: (0, i, 0)),
          pl.BlockSpec((1, _BLK, D), lambda i: (1, i, 0)),
          pl.BlockSpec((1, _BLK, D), lambda i: (0, i, 0)),
          pl.BlockSpec((1, _BLK, D), lambda i: (1, i, 0)),
          _full_spec((D + 16, D)),
          _full_spec((1, D)),
          _full_spec((2 * D, D)),
          _full_spec((1, D)),
      ],
      out_specs=_row_spec(D),
      out_shape=jax.ShapeDtypeStruct((N, D), jnp.float32),
  )(h, s_parts, s_parts, ea_parts, ea_parts, wm, bm, wu, bu)


def _dense_final(h, s_parts, ea_parts, wm, bm, wu, bu, wp1, bp1, wp2, bp2):
  return pl.pallas_call(
      _dense_final_body,
      grid=(_GRID,),
      in_specs=[
          _row_spec(D),
          pl.BlockSpec((1, _BLK, D), lambda i: (0, i, 0)),
          pl.BlockSpec((1, _BLK, D), lambda i: (1, i, 0)),
          pl.BlockSpec((1, _BLK, D), lambda i: (0, i, 0)),
          pl.BlockSpec((1, _BLK, D), lambda i: (1, i, 0)),
          _full_spec((D + 16, D)),
          _full_spec((1, D)),
          _full_spec((2 * D, D)),
          _full_spec((1, D)),
          _full_spec((D, D)),
          _full_spec((1, D)),
          _full_spec((D, D)),
          _full_spec((1, D)),
      ],
      out_specs=[_full_spec((1, D)), _full_spec((1, D))],
      out_shape=[jax.ShapeDtypeStruct((1, D), jnp.float32),
                 jax.ShapeDtypeStruct((1, D), jnp.float32)],
      scratch_shapes=[pltpu.VMEM((1, D), jnp.float32)],
  )(h, s_parts, s_parts, ea_parts, ea_parts, wm, bm, wu, bu,
    wp1, bp1, wp2, bp2)


def kernel(x, edge_index, edge_attr,
           W_msg0, b_msg0, W_upd0, b_upd0,
           W_msg1, b_msg1, W_upd1, b_upd1,
           W_msg2, b_msg2, W_upd2, b_upd2,
           W_p1, b_p1, W_p2, b_p2):
  src = edge_index[0].astype(jnp.int32)
  dst = edge_index[1].astype(jnp.int32)
  pad = EPAD - E
  ar = lax.iota(jnp.int32, pad)
  src_p = jnp.concatenate([src, ar % N])
  dst_p = jnp.concatenate([dst, N + ar % (NPAD - N)])
  packed3 = (src_p | (dst_p << 16)).reshape(NW, NCH, CHUNK)
  ea3 = edge_attr.reshape(ECH, 16, CHUNK)

  bm0 = b_msg0.reshape(1, D); bu0 = b_upd0.reshape(1, D)
  bm1 = b_msg1.reshape(1, D); bu1 = b_upd1.reshape(1, D)
  bm2 = b_msg2.reshape(1, D); bu2 = b_upd2.reshape(1, D)
  bp1 = b_p1.reshape(1, D); bp2 = b_p2.reshape(1, D)

  s_parts, ea_parts = _spmm_l0(packed3, x, ea3)
  h = _dense_mid(x, s_parts, ea_parts, W_msg0, bm0, W_upd0, bu0)
  (s_parts2,) = _spmm(packed3, h)
  h = _dense_mid(h, s_parts2, ea_parts, W_msg1, bm1, W_upd1, bu1)
  (s_parts3,) = _spmm(packed3, h)
  z, gr = _dense_final(h, s_parts3, ea_parts, W_msg2, bm2, W_upd2, bu2,
                       W_p1, bp1, W_p2, bp2)
  return (z.reshape(D), gr.reshape(D))


# speedup vs baseline: 1.1793x; 1.0048x over previous
import jax
import jax.numpy as jnp
from jax import lax
from jax.experimental import pallas as pl
from jax.experimental.pallas import tpu as pltpu
from jax.experimental.pallas import tpu_sc as plsc

N = 10000
E = 320000
D = 128
NC = 2
NS = 16
NW = NC * NS
CHUNK = 128
NCH = 80
EPT = NCH * CHUNK
EPAD = NW * EPT
NPAD = 10240
ZROWS = NPAD // NS


_MESH = plsc.VectorSubcoreMesh(core_axis_name="c", subcore_axis_name="s")


def _zero_fill(buf, width):
  z16 = jnp.zeros((16,), jnp.float32)

  def zrow(i, _):
    for k in range(width // 16):
      buf[i, pl.ds(k * 16, 16)] = z16
    return 0

  lax.fori_loop(0, CHUNK, zrow, 0)


def _unpack(pk, sidx, didx, j, t):
  m = jnp.full((16,), 0xFFFF, jnp.int32)
  for k in range(CHUNK // 16):
    v = pk[j, pl.ds(k * 16, 16)]
    sidx[t, pl.ds(k * 16, 16)] = v & m
    didx[t, pl.ds(k * 16, 16)] = lax.shift_right_logical(v, 16)


def _zero_acc(src_rows, acc, s):
  _zero_fill(src_rows, D)
  for k in range(ZROWS // CHUNK):
    pltpu.sync_copy(src_rows, acc.at[pl.ds(s * ZROWS + k * CHUNK, CHUNK)])
  plsc.subcore_barrier()


def _spmm_body(pk_h, x_h, s_out, pk, sidx, didx, rows0, rows1, acc, gsem, ssem):
  c = lax.axis_index("c")
  s = lax.axis_index("s")
  tid = c * NS + s

  pltpu.sync_copy(pk_h.at[tid], pk)
  _zero_acc(rows0, acc, s)

  _unpack(pk, sidx, didx, 0, 0)
  pltpu.async_copy(x_h.at[sidx.at[0]], rows0, gsem)
  bufs = (rows0, rows1)

  def pair(g, _):
    for t in range(2):
      j = 2 * g + t
      rb = bufs[t]
      ob = bufs[1 - t]
      pltpu.make_async_copy(x_h.at[sidx.at[t]], rb, gsem).wait()

      @pl.when(j >= 1)
      def _wait_prev():
        pltpu.make_async_copy(ob, acc.at[didx.at[1 - t]], ssem).wait()

      pltpu.async_copy(rb, acc.at[didx.at[t]], ssem, add=True)

      @pl.when(j + 1 < NCH)
      def _next_gather():
        _unpack(pk, sidx, didx, j + 1, 1 - t)
        pltpu.async_copy(x_h.at[sidx.at[1 - t]], ob, gsem)
    return 0

  lax.fori_loop(0, NCH // 2, pair, 0)
  pltpu.make_async_copy(rows1, acc.at[didx.at[1]], ssem).wait()
  plsc.subcore_barrier()

  pltpu.sync_copy(acc.at[pl.ds(s * ZROWS, ZROWS)],
                  s_out.at[c, pl.ds(s * ZROWS, ZROWS)])


_spmm = pl.kernel(
    _spmm_body, mesh=_MESH,
    out_type=[jax.ShapeDtypeStruct((NC, NPAD, D), jnp.float32)],
    scratch_types=[
        pltpu.VMEM((NCH, CHUNK), jnp.int32),
        pltpu.VMEM((2, CHUNK), jnp.int32),
        pltpu.VMEM((2, CHUNK), jnp.int32),
        pltpu.VMEM((CHUNK, D), jnp.float32),
        pltpu.VMEM((CHUNK, D), jnp.float32),
        pltpu.VMEM_SHARED((NPAD, D), jnp.float32),
        pltpu.SemaphoreType.DMA,
        pltpu.SemaphoreType.DMA,
    ])


def _unpack_d(pk, didx, j, t):
  for k in range(CHUNK // 16):
    v = pk[j, pl.ds(k * 16, 16)]
    didx[t, pl.ds(k * 16, 16)] = lax.shift_right_logical(v, 16)


ECH = E // CHUNK


def _l0_body(pk_h, x_h, ea_h, s_out, ea_out,
             pk, sidx, didx, eb0, eb1, rows0, rows1, acc, gsem, ssem):
  c = lax.axis_index("c")
  s = lax.axis_index("s")
  tid = c * NS + s

  pltpu.sync_copy(pk_h.at[tid], pk)

  _zero_acc(rows0, acc, s)
  _zero_fill(rows1, D)
  one0 = jnp.where(lax.iota(jnp.int32, 16) == 0,
                   jnp.float32(1), jnp.float32(0))

  def fill_tail(i, _):
    rows0[i, pl.ds(16, 16)] = one0
    rows1[i, pl.ds(16, 16)] = one0
    return 0

  lax.fori_loop(0, CHUNK, fill_tail, 0)

  nj = jnp.minimum(ECH - tid * NCH, NCH)
  _unpack_d(pk, didx, 0, 0)
  pltpu.async_copy(ea_h.at[tid * NCH], eb0, gsem)
  ebufs = (eb0, eb1)
  rbufs = (rows0, rows1)

  def ea_pair(g, _):
    for t in range(2):
      j = 2 * g + t
      eb = ebufs[t]
      rb = rbufs[t]
      pltpu.make_async_copy(ea_h.at[tid * NCH + j], eb, gsem).wait()

      @pl.when(j >= 2)
      def _wait_prev():
        pltpu.make_async_copy(rb, acc.at[didx.at[t]], ssem).wait()

      @pl.when(j + 1 < nj)
      def _next_load():
        pltpu.async_copy(ea_h.at[tid * NCH + j + 1], ebufs[1 - t], gsem)

      _unpack_d(pk, didx, j, t)

      def expand(r, _):
        for q in range(8):
          rb[r * 8 + q, pl.ds(0, 16)] = eb[r, pl.ds(q * 16, 16)]
        return 0

      lax.fori_loop(0, 16, expand, 0)
      pltpu.async_copy(rb, acc.at[didx.at[t]], ssem, add=True)
    return 0

  lax.fori_loop(0, nj // 2, ea_pair, 0)
  pltpu.make_async_copy(rows0, acc.at[didx.at[0]], ssem).wait()
  pltpu.make_async_copy(rows1, acc.at[didx.at[1]], ssem).wait()
  plsc.subcore_barrier()
  pltpu.sync_copy(acc.at[pl.ds(s * ZROWS, ZROWS)],
                  ea_out.at[c, pl.ds(s * ZROWS, ZROWS)])

  _zero_acc(rows0, acc, s)
  _unpack(pk, sidx, didx, 0, 0)
  pltpu.async_copy(x_h.at[sidx.at[0]], rows0, gsem)

  def pair(g, _):
    for t in range(2):
      j = 2 * g + t
      rb = rbufs[t]
      ob = rbufs[1 - t]
      pltpu.make_async_copy(x_h.at[sidx.at[t]], rb, gsem).wait()

      @pl.when(j >= 1)
      def _wait_prev():
        pltpu.make_async_copy(ob, acc.at[didx.at[1 - t]], ssem).wait()

      pltpu.async_copy(rb, acc.at[didx.at[t]], ssem, add=True)

      @pl.when(j + 1 < NCH)
      def _next_gather():
        _unpack(pk, sidx, didx, j + 1, 1 - t)
        pltpu.async_copy(x_h.at[sidx.at[1 - t]], ob, gsem)
    return 0

  lax.fori_loop(0, NCH // 2, pair, 0)
  pltpu.make_async_copy(rows1, acc.at[didx.at[1]], ssem).wait()
  plsc.subcore_barrier()
  pltpu.sync_copy(acc.at[pl.ds(s * ZROWS, ZROWS)],
                  s_out.at[c, pl.ds(s * ZROWS, ZROWS)])


_spmm_l0 = pl.kernel(
    _l0_body, mesh=_MESH,
    out_type=[jax.ShapeDtypeStruct((NC, NPAD, D), jnp.float32),
              jax.ShapeDtypeStruct((NC, NPAD, D), jnp.float32)],
    scratch_types=[
        pltpu.VMEM((NCH, CHUNK), jnp.int32),
        pltpu.VMEM((2, CHUNK), jnp.int32),
        pltpu.VMEM((2, CHUNK), jnp.int32),
        pltpu.VMEM((16, CHUNK), jnp.float32),
        pltpu.VMEM((16, CHUNK), jnp.float32),
        pltpu.VMEM((CHUNK, D), jnp.float32),
        pltpu.VMEM((CHUNK, D), jnp.float32),
        pltpu.VMEM_SHARED((NPAD, D), jnp.float32),
        pltpu.SemaphoreType.DMA,
        pltpu.SemaphoreType.DMA,
    ])


_BLK = 1000
_GRID = N // _BLK


def _dense_mid_body(h, s0, s1, e0, e1, wm, bm, wu, bu, out):
  sp = s0[0] + s1[0]
  eas = e0[0] + e1[0]
  agg = (jnp.dot(sp, wm[:D, :], preferred_element_type=jnp.float32)
         + jnp.dot(eas[:, :16], wm[D:, :], preferred_element_type=jnp.float32)
         + eas[:, 16:17] * bm[...])
  upd = (jnp.dot(h[...], wu[:D, :], preferred_element_type=jnp.float32)
         + jnp.dot(agg, wu[D:, :], preferred_element_type=jnp.float32)
         + bu[...])
  out[...] = jnp.maximum(upd, 0.0)


def _dense_final_body(h, s0, s1, e0, e1, wm, bm, wu, bu, wp1, bp1, wp2, bp2,
                      z_out, gr_out, accum):
  i = pl.program_id(0)

  @pl.when(i == 0)
  def _init():
    accum[...] = jnp.zeros_like(accum)

  sp = s0[0] + s1[0]
  eas = e0[0] + e1[0]
  agg = (jnp.dot(sp, wm[:D, :], preferred_element_type=jnp.float32)
         + jnp.dot(eas[:, :16], wm[D:, :], preferred_element_type=jnp.float32)
         + eas[:, 16:17] * bm[...])
  upd = (jnp.dot(h[...], wu[:D, :], preferred_element_type=jnp.float32)
         + jnp.dot(agg, wu[D:, :], preferred_element_type=jnp.float32)
         + bu[...])
  hb = jnp.maximum(upd, 0.0)
  accum[...] += jnp.sum(hb, axis=0, keepdims=True)

  @pl.when(i == _GRID - 1)
  def _readout():
    gr = accum[...] * (1.0 / N)
    gr_out[...] = gr
    t = jnp.maximum(
        jnp.dot(gr, wp1[...], preferred_element_type=jnp.float32) + bp1[...],
        0.0)
    z_out[...] = jnp.dot(t, wp2[...], preferred_element_type=jnp.float32) + bp2[...]


def _row_spec(width):
  return pl.BlockSpec((_BLK, width), lambda i: (i, 0))


def _full_spec(shape):
  return pl.BlockSpec(shape, lambda i: tuple(0 for _ in shape))


def _dense_mid(h, s_parts, ea_parts, wm, bm, wu, bu):
  return pl.pallas_call(
      _dense_mid_body,
      grid=(_GRID,),
      in_specs=[
          _row_spec(D),
          pl.BlockSpec((1, _BLK, D), lambda i: (0, i, 0)),
          pl.BlockSpec((1, _BLK, D), lambda i: (1, i, 0)),
          pl.BlockSpec((1, _BLK, D), lambda i: (0, i, 0)),
          pl.BlockSpec((1, _BLK, D), lambda i: (1, i, 0)),
          _full_spec((D + 16, D)),
          _full_spec((1, D)),
          _full_spec((2 * D, D)),
          _full_spec((1, D)),
      ],
      out_specs=_row_spec(D),
      out_shape=jax.ShapeDtypeStruct((N, D), jnp.float32),
  )(h, s_parts, s_parts, ea_parts, ea_parts, wm, bm, wu, bu)


def _dense_final(h, s_parts, ea_parts, wm, bm, wu, bu, wp1, bp1, wp2, bp2):
  return pl.pallas_call(
      _dense_final_body,
      grid=(_GRID,),
      in_specs=[
          _row_spec(D),
          pl.BlockSpec((1, _BLK, D), lambda i: (0, i, 0)),
          pl.BlockSpec((1, _BLK, D), lambda i: (1, i, 0)),
          pl.BlockSpec((1, _BLK, D), lambda i: (0, i, 0)),
          pl.BlockSpec((1, _BLK, D), lambda i: (1, i, 0)),
          _full_spec((D + 16, D)),
          _full_spec((1, D)),
          _full_spec((2 * D, D)),
          _full_spec((1, D)),
          _full_spec((D, D)),
          _full_spec((1, D)),
          _full_spec((D, D)),
          _full_spec((1, D)),
      ],
      out_specs=[_full_spec((1, D)), _full_spec((1, D))],
      out_shape=[jax.ShapeDtypeStruct((1, D), jnp.float32),
                 jax.ShapeDtypeStruct((1, D), jnp.float32)],
      scratch_shapes=[pltpu.VMEM((1, D), jnp.float32)],
  )(h, s_parts, s_parts, ea_parts, ea_parts, wm, bm, wu, bu,
    wp1, bp1, wp2, bp2)


def kernel(x, edge_index, edge_attr,
           W_msg0, b_msg0, W_upd0, b_upd0,
           W_msg1, b_msg1, W_upd1, b_upd1,
           W_msg2, b_msg2, W_upd2, b_upd2,
           W_p1, b_p1, W_p2, b_p2):
  src = edge_index[0].astype(jnp.int32)
  dst = edge_index[1].astype(jnp.int32)
  pad = EPAD - E
  ar = lax.iota(jnp.int32, pad)
  src_p = jnp.concatenate([src, ar % N])
  dst_p = jnp.concatenate([dst, N + ar % (NPAD - N)])
  packed3 = (src_p | (dst_p << 16)).reshape(NW, NCH, CHUNK)
  ea3 = edge_attr.reshape(ECH, 16, CHUNK)

  bm0 = b_msg0.reshape(1, D); bu0 = b_upd0.reshape(1, D)
  bm1 = b_msg1.reshape(1, D); bu1 = b_upd1.reshape(1, D)
  bm2 = b_msg2.reshape(1, D); bu2 = b_upd2.reshape(1, D)
  bp1 = b_p1.reshape(1, D); bp2 = b_p2.reshape(1, D)

  s_parts, ea_parts = _spmm_l0(packed3, x, ea3)
  h = _dense_mid(x, s_parts, ea_parts, W_msg0, bm0, W_upd0, bu0)
  (s_parts2,) = _spmm(packed3, h)
  h = _dense_mid(h, s_parts2, ea_parts, W_msg1, bm1, W_upd1, bu1)
  (s_parts3,) = _spmm(packed3, h)
  z, gr = _dense_final(h, s_parts3, ea_parts, W_msg2, bm2, W_upd2, bu2,
                       W_p1, bp1, W_p2, bp2)
  return (z.reshape(D), gr.reshape(D))
